# baseline (speedup 1.0000x reference)
import jax
import jax.numpy as jnp
from jax import lax
from jax.experimental import pallas as pl
from jax.experimental.pallas import tpu as pltpu
from jax.experimental.pallas import tpu_sc as plsc

_NUM_FEATURES = 100
_EMBED_DIM = 32
_BATCH = 16384
_ROW = _NUM_FEATURES * _EMBED_DIM
_LANES = 16
_VECS = _ROW // _LANES

_NC = 2
_NS = 16
_NW = _NC * _NS
_RPW = _BATCH // _NW
_C = 8
_NCH = _RPW // _C
_NP = _NCH // 2


def _sc_body(x_hbm, tab_hbm, out_hbm, tab_v, bi, bo, sin0, sin1, so0, so1):
    sins = (sin0, sin1)
    souts = (so0, so1)
    wid = lax.axis_index("s") * _NC + lax.axis_index("c")
    base = wid * _RPW
    pltpu.sync_copy(tab_hbm, tab_v)

    def start_in(g, b):
        r0 = base + g * _C
        pltpu.async_copy(x_hbm.at[pl.ds(r0, _C)], bi.at[b], sins[b])

    def wait_in(b):
        pltpu.make_async_copy(x_hbm.at[pl.ds(0, _C)], bi.at[b], sins[b]).wait()

    def start_out(g, b):
        r0 = base + g * _C
        pltpu.async_copy(bo.at[b], out_hbm.at[pl.ds(r0, _C)], souts[b])

    def wait_out(b):
        pltpu.make_async_copy(bo.at[b], out_hbm.at[pl.ds(0, _C)], souts[b]).wait()

    def compute(b):
        def jloop(j, c2):
            j16 = j * _LANES
            t = tab_v[pl.ds(j16, _LANES)]
            for r in range(_C):
                bo[b, r, pl.ds(j16, _LANES)] = bi[b, r, pl.ds(j16, _LANES)] + t
            return c2

        lax.fori_loop(0, _VECS, jloop, 0, unroll=2)

    start_in(0, 0)
    start_in(1, 1)
    for b in range(2):
        wait_in(b)
        compute(b)
        start_out(b, b)
        start_in(b + 2, b)

    def pair(p, carry):
        for b in range(2):
            g = p * 2 + b
            wait_in(b)
            wait_out(b)
            compute(b)
            start_out(g, b)
            nxt = jnp.minimum(g + 2, _NCH - 1)
            start_in(nxt, b)
        return carry

    lax.fori_loop(1, _NP, pair, 0)

    wait_in(0)
    wait_in(1)
    wait_out(0)
    wait_out(1)


def kernel(inputs, column_table):
    x = inputs.reshape(_BATCH, _ROW)
    tab = column_table.reshape(_ROW)
    mesh = plsc.VectorSubcoreMesh(core_axis_name="c", subcore_axis_name="s")
    out = pl.kernel(
        _sc_body,
        out_type=jax.ShapeDtypeStruct((_BATCH, _ROW), jnp.float32),
        mesh=mesh,
        scratch_types=[
            pltpu.VMEM((_ROW,), jnp.float32),
            pltpu.VMEM((2, _C, _ROW), jnp.float32),
            pltpu.VMEM((2, _C, _ROW), jnp.float32),
            pltpu.SemaphoreType.DMA,
            pltpu.SemaphoreType.DMA,
            pltpu.SemaphoreType.DMA,
            pltpu.SemaphoreType.DMA,
        ],
    )(x, tab)
    return out.reshape(_BATCH, _NUM_FEATURES, _EMBED_DIM)


# speedup vs baseline: 1.3039x; 1.3039x over previous
import jax
import jax.numpy as jnp
from jax import lax
from jax.experimental import pallas as pl
from jax.experimental.pallas import tpu as pltpu
from jax.experimental.pallas import tpu_sc as plsc

_NUM_FEATURES = 100
_EMBED_DIM = 32
_BATCH = 16384
_ROW = _NUM_FEATURES * _EMBED_DIM
_LANES = 16
_VECS = _ROW // _LANES

_NC = 2
_NS = 16
_NW = _NC * _NS
_RPW = _BATCH // _NW
_C = 8
_U = 10
_NCH = _RPW // _C
_NP = _NCH // 2


def _sc_body(x_hbm, tab_hbm, out_hbm, tab_v, bi, bo, sin0, sin1, so0, so1):
    sins = (sin0, sin1)
    souts = (so0, so1)
    wid = lax.axis_index("s") * _NC + lax.axis_index("c")
    base = wid * _RPW
    pltpu.sync_copy(tab_hbm, tab_v)

    def start_in(g, b):
        r0 = base + g * _C
        pltpu.async_copy(x_hbm.at[pl.ds(r0, _C)], bi.at[b], sins[b])

    def wait_in(b):
        pltpu.make_async_copy(x_hbm.at[pl.ds(0, _C)], bi.at[b], sins[b]).wait()

    def start_out(g, b):
        r0 = base + g * _C
        pltpu.async_copy(bo.at[b], out_hbm.at[pl.ds(r0, _C)], souts[b])

    def wait_out(b):
        pltpu.make_async_copy(bo.at[b], out_hbm.at[pl.ds(0, _C)], souts[b]).wait()

    def compute(b):
        for jb in range(_VECS // _U):
            ts = [tab_v[pl.ds((jb * _U + u) * _LANES, _LANES)] for u in range(_U)]

            def rbody(r, c2):
                for u in range(_U):
                    off = (jb * _U + u) * _LANES
                    bo[b, r, pl.ds(off, _LANES)] = (
                        bi[b, r, pl.ds(off, _LANES)] + ts[u]
                    )
                return c2

            lax.fori_loop(0, _C, rbody, 0, unroll=2)

    start_in(0, 0)
    start_in(1, 1)
    for b in range(2):
        wait_in(b)
        compute(b)
        start_out(b, b)
        start_in(b + 2, b)

    def pair(p, carry):
        for b in range(2):
            g = p * 2 + b
            wait_in(b)
            wait_out(b)
            compute(b)
            start_out(g, b)
            nxt = jnp.minimum(g + 2, _NCH - 1)
            start_in(nxt, b)
        return carry

    lax.fori_loop(1, _NP, pair, 0)

    wait_in(0)
    wait_in(1)
    wait_out(0)
    wait_out(1)


def kernel(inputs, column_table):
    x = inputs.reshape(_BATCH, _ROW)
    tab = column_table.reshape(_ROW)
    mesh = plsc.VectorSubcoreMesh(core_axis_name="c", subcore_axis_name="s")
    out = pl.kernel(
        _sc_body,
        out_type=jax.ShapeDtypeStruct((_BATCH, _ROW), jnp.float32),
        mesh=mesh,
        scratch_types=[
            pltpu.VMEM((_ROW,), jnp.float32),
            pltpu.VMEM((2, _C, _ROW), jnp.float32),
            pltpu.VMEM((2, _C, _ROW), jnp.float32),
            pltpu.SemaphoreType.DMA,
            pltpu.SemaphoreType.DMA,
            pltpu.SemaphoreType.DMA,
            pltpu.SemaphoreType.DMA,
        ],
    )(x, tab)
    return out.reshape(_BATCH, _NUM_FEATURES, _EMBED_DIM)
